# Initial kernel scaffold; baseline (speedup 1.0000x reference)
import jax
import jax.numpy as jnp
from jax.experimental import pallas as pl


def kernel(feature_s, xyz_s, feature_t, xyz_t, W_s, b_s, gamma_s, beta_s, rmean_s, rvar_s, W_t, b_t, gamma_t, beta_t, rmean_t, rvar_t):
    raise NotImplementedError("write your pallas kernel here")


# speedup vs baseline: 22.3327x; 22.3327x over previous
import functools

import jax
import jax.numpy as jnp
import numpy as np
from jax import lax
from jax.experimental import pallas as pl
from jax.experimental.pallas import tpu as pltpu
from jax.experimental.pallas import tpu_sc as plsc

_K = 12
_NPOINT = 32
_BN_EPS = 1e-5
_N = 32768
_INT_MAX = np.int32(2**31 - 1)


def _fps_knn_body(xt8_ref, xtl_ref, xsl_ref, idxt_ref, idxs_ref,
                  dist_ref, d_ref):
    b = pl.program_id(0)

    x8 = xt8_ref[0, 0]
    y8 = xt8_ref[0, 1]
    z8 = xt8_ref[0, 2]
    iota8 = (lax.broadcasted_iota(jnp.int32, (8, 4096), 0) * 4096
             + lax.broadcasted_iota(jnp.int32, (8, 4096), 1))
    iota32 = lax.broadcasted_iota(jnp.int32, (_NPOINT, 1), 0)

    dist_ref[...] = jnp.full((8, 4096), 1e10, dtype=jnp.float32)

    def fps_step(i, carry):
        far, cxv, cyv, czv = carry
        sel = iota8 == far
        cx = jnp.sum(jnp.where(sel, x8, 0.0))
        cy = jnp.sum(jnp.where(sel, y8, 0.0))
        cz = jnp.sum(jnp.where(sel, z8, 0.0))
        cxv = jnp.where(iota32 == i, cx, cxv)
        cyv = jnp.where(iota32 == i, cy, cyv)
        czv = jnp.where(iota32 == i, cz, czv)
        dx = x8 - cx
        dy = y8 - cy
        dz = z8 - cz
        d = (dx * dx + dy * dy) + dz * dz
        dist = jnp.minimum(dist_ref[...], d)
        dist_ref[...] = dist
        m = jnp.max(dist)
        far_next = jnp.min(jnp.where(dist == m, iota8, _INT_MAX))
        return far_next, cxv, cyv, czv

    zeros32 = jnp.zeros((_NPOINT, 1), dtype=jnp.float32)
    _, cxv, cyv, czv = lax.fori_loop(
        0, _NPOINT, fps_step, (jnp.int32(0), zeros32, zeros32, zeros32))

    c2 = (cxv * cxv + cyv * cyv) + czv * czv
    bcx = cxv.astype(jnp.bfloat16).astype(jnp.float32)
    bcy = cyv.astype(jnp.bfloat16).astype(jnp.float32)
    bcz = czv.astype(jnp.bfloat16).astype(jnp.float32)
    iota_l = lax.broadcasted_iota(jnp.int32, (_NPOINT, _N), 1)
    row_off = b * _N

    def knn(xl_ref, out_ref):
        xl = xl_ref[0, 0:1, :]
        yl = xl_ref[0, 1:2, :]
        zl = xl_ref[0, 2:3, :]
        n2 = (xl * xl + yl * yl) + zl * zl
        bx = xl.astype(jnp.bfloat16).astype(jnp.float32)
        by = yl.astype(jnp.bfloat16).astype(jnp.float32)
        bz = zl.astype(jnp.bfloat16).astype(jnp.float32)
        dot = (bcx * bx + bcy * by) + bcz * bz
        d_ref[...] = (c2 + n2) - 2.0 * dot
        for j in range(_K):
            dv = d_ref[...]
            m = jnp.min(dv, axis=1, keepdims=True)
            idx = jnp.min(jnp.where(dv == m, iota_l, _INT_MAX),
                          axis=1, keepdims=True)
            out_ref[0, :, j:j + 1] = idx + row_off
            if j + 1 < _K:
                d_ref[...] = jnp.where(iota_l == idx, jnp.inf, dv)

    knn(xtl_ref, idxt_ref)
    knn(xsl_ref, idxs_ref)


def _fps_knn(xt8, xtl, xsl):
    return pl.pallas_call(
        _fps_knn_body,
        grid=(2,),
        in_specs=[
            pl.BlockSpec((1, 3, 8, 4096), lambda b: (b, 0, 0, 0)),
            pl.BlockSpec((1, 3, _N), lambda b: (b, 0, 0)),
            pl.BlockSpec((1, 3, _N), lambda b: (b, 0, 0)),
        ],
        out_specs=[
            pl.BlockSpec((1, _NPOINT, _K), lambda b: (b, 0, 0)),
            pl.BlockSpec((1, _NPOINT, _K), lambda b: (b, 0, 0)),
        ],
        out_shape=[
            jax.ShapeDtypeStruct((2, _NPOINT, _K), jnp.int32),
            jax.ShapeDtypeStruct((2, _NPOINT, _K), jnp.int32),
        ],
        scratch_shapes=[
            pltpu.VMEM((8, 4096), jnp.float32),
            pltpu.VMEM((_NPOINT, _N), jnp.float32),
        ],
    )(xt8, xtl, xsl)


def _sc_gather(ft_flat, fs_flat, idxt_flat, idxs_flat, dt, ds, nrows):
    info = plsc.get_sparse_core_info()
    nw = info.num_cores * info.num_subcores
    per_w = nrows // nw
    mesh = plsc.VectorSubcoreMesh(core_axis_name="c", subcore_axis_name="s")

    @functools.partial(
        pl.kernel,
        out_type=[
            jax.ShapeDtypeStruct((nrows, dt), jnp.float32),
            jax.ShapeDtypeStruct((nrows, ds), jnp.float32),
        ],
        mesh=mesh,
        scratch_types=[
            pltpu.VMEM((per_w,), jnp.int32),
            pltpu.VMEM((per_w, dt), jnp.float32),
            pltpu.VMEM((per_w,), jnp.int32),
            pltpu.VMEM((per_w, ds), jnp.float32),
            pltpu.SemaphoreType.DMA,
            pltpu.SemaphoreType.DMA,
        ],
    )
    def gather_k(ft_hbm, fs_hbm, idxt_hbm, idxs_hbm, outt_hbm, outs_hbm,
                 idxt_v, rowst_v, idxs_v, rowss_v, semt, sems):
        wid = lax.axis_index("s") * info.num_cores + lax.axis_index("c")
        base = wid * per_w
        pltpu.sync_copy(idxt_hbm.at[pl.ds(base, per_w)], idxt_v)
        pltpu.sync_copy(idxs_hbm.at[pl.ds(base, per_w)], idxs_v)
        cpt = pltpu.async_copy(ft_hbm.at[idxt_v], rowst_v, semt)
        cps = pltpu.async_copy(fs_hbm.at[idxs_v], rowss_v, sems)
        cpt.wait()
        cps.wait()
        pltpu.sync_copy(rowst_v, outt_hbm.at[pl.ds(base, per_w)])
        pltpu.sync_copy(rowss_v, outs_hbm.at[pl.ds(base, per_w)])

    return gather_k(ft_flat, fs_flat, idxt_flat, idxs_flat)


def _conv_pool_body(gt_ref, gs_ref, wt_ref, ws_ref, kt1_ref, kt2_ref,
                    ks1_ref, ks2_ref, outt_ref, outs_ref):
    def one(g_ref, w_ref, k1_ref, k2_ref, out_ref):
        g = g_ref[...].astype(jnp.bfloat16)
        w = w_ref[...]
        y = lax.dot_general(g, w, (((1,), (1,)), ((), ())),
                            preferred_element_type=jnp.float32)
        y = y * k1_ref[...] + k2_ref[...]
        y = jnp.maximum(y, 0.0)
        acc = y[0:64]
        for j in range(1, _K):
            acc = jnp.maximum(acc, y[j * 64:(j + 1) * 64])
        out_ref[...] = acc

    one(gt_ref, wt_ref, kt1_ref, kt2_ref, outt_ref)
    one(gs_ref, ws_ref, ks1_ref, ks2_ref, outs_ref)


def _conv_pool(gt, gs, wt_bf, ws_bf, kt1, kt2, ks1, ks2):
    n_out = 64
    return pl.pallas_call(
        _conv_pool_body,
        out_shape=[
            jax.ShapeDtypeStruct((n_out, wt_bf.shape[0]), jnp.float32),
            jax.ShapeDtypeStruct((n_out, ws_bf.shape[0]), jnp.float32),
        ],
    )(gt, gs, wt_bf, ws_bf, kt1, kt2, ks1, ks2)


def kernel(feature_s, xyz_s, feature_t, xyz_t,
           W_s, b_s, gamma_s, beta_s, rmean_s, rvar_s,
           W_t, b_t, gamma_t, beta_t, rmean_t, rvar_t):
    B, N, _ = xyz_t.shape
    nrows = B * _NPOINT * _K

    xt = jnp.transpose(xyz_t, (0, 2, 1))
    xs = jnp.transpose(xyz_s, (0, 2, 1))
    xt8 = xt.reshape(B, 3, 8, N // 8)

    idx_t, idx_s = _fps_knn(xt8, xt, xs)

    idxt_flat = jnp.transpose(idx_t, (2, 0, 1)).reshape(nrows)
    idxs_flat = jnp.transpose(idx_s, (2, 0, 1)).reshape(nrows)

    dt = feature_t.shape[-1]
    ds = feature_s.shape[-1]
    gt, gs = _sc_gather(feature_t.reshape(B * N, dt),
                        feature_s.reshape(B * N, ds),
                        idxt_flat, idxs_flat, dt, ds, nrows)

    inv_t = 1.0 / jnp.sqrt(rvar_t + _BN_EPS)
    inv_s = 1.0 / jnp.sqrt(rvar_s + _BN_EPS)
    kt1 = (gamma_t * inv_t)[None, :]
    kt2 = ((b_t - rmean_t) * gamma_t * inv_t + beta_t)[None, :]
    ks1 = (gamma_s * inv_s)[None, :]
    ks2 = ((b_s - rmean_s) * gamma_s * inv_s + beta_s)[None, :]

    out_t, out_s = _conv_pool(gt, gs,
                              W_t.astype(jnp.bfloat16),
                              W_s.astype(jnp.bfloat16),
                              kt1, kt2, ks1, ks2)

    d_out = W_s.shape[0]
    return (out_s.reshape(B, _NPOINT, d_out), out_t.reshape(B, _NPOINT, d_out))
